# baseline (speedup 1.0000x reference)
import functools

import jax
import jax.numpy as jnp
from jax import lax
from jax.experimental import pallas as pl
from jax.experimental.pallas import tpu as pltpu
from jax.experimental.pallas import tpu_sc as plsc

_BATCH = 16384
_FIELDS = 26
_DIM = 64
_V = 1000000
_B = _BATCH * _FIELDS
_NC = 2
_NS = 16
_NW = _NC * _NS
_VB = 999936
_NBLK = _VB // 128
_BPW = (_NBLK + _NW - 1) // _NW
_NCH = _B // 128
_CPW = _NCH // _NW

_mesh = plsc.VectorSubcoreMesh(core_axis_name="c", subcore_axis_name="s")
_tiled = pltpu.CompilerParams(use_tc_tiling_on_sc=True, needs_layout_passes=False)


def _iota16():
    return lax.iota(jnp.int32, 16)


@functools.partial(
    pl.kernel,
    mesh=_mesh,
    compiler_params=_tiled,
    out_type=jax.ShapeDtypeStruct((_V // 2, 128), jnp.float32),
    scratch_types=[
        pltpu.VMEM((8, 8, 128), jnp.float32),
        pltpu.VMEM((64, 128), jnp.float32),
        pltpu.SemaphoreType.DMA,
    ],
)
def _fmt(wT_hbm, wtail_hbm, tbl_hbm, staged, blk, sem):
    wid = lax.axis_index("s") * _NC + lax.axis_index("c")
    iota = _iota16()
    dgv = [jnp.bitwise_and(16 * k + iota, 63) >> 3 for k in range(8)]
    dsv = [jnp.bitwise_and(16 * k + iota, 63) & 7 for k in range(8)]

    def block(i, carry):
        rb = wid * _BPW + i

        @pl.when(rb < _NBLK)
        def _():
            for dg in range(8):
                pltpu.async_copy(
                    wT_hbm.at[pl.ds(8 * dg, 8), pl.ds(128 * rb, 128)],
                    staged.at[dg], sem,
                )
            for dg in range(8):
                pltpu.make_async_copy(
                    wT_hbm.at[pl.ds(8 * dg, 8), pl.ds(128 * rb, 128)],
                    staged.at[dg], sem,
                ).wait()

            def row(jj, c2):
                for k in range(8):
                    rv = jnp.zeros((16,), jnp.int32) + (2 * jj + (1 if k >= 4 else 0))
                    v = plsc.load_gather(staged, [dgv[k], dsv[k], rv])
                    blk[jj, pl.ds(16 * k, 16)] = v
                return c2

            lax.fori_loop(0, 64, row, 0)
            pltpu.sync_copy(blk, tbl_hbm.at[pl.ds(64 * rb, 64)])

        return carry

    lax.fori_loop(0, _BPW, block, 0)

    @pl.when(wid == 0)
    def _():
        for q in range(8):
            pltpu.sync_copy(wtail_hbm.at[pl.ds(8 * q, 8)], staged.at[q])

        def trow(t2, c2):
            for k in range(8):
                rv = jnp.zeros((16,), jnp.int32) + (2 * t2 + (1 if k >= 4 else 0))
                dv = jnp.bitwise_and(16 * k + iota, 63)
                v = plsc.load_gather(staged, [rv >> 3, rv & 7, dv])
                blk[t2, pl.ds(16 * k, 16)] = v
            return c2

        lax.fori_loop(0, 32, trow, 0)
        pltpu.sync_copy(
            blk.at[pl.ds(0, 32)], tbl_hbm.at[pl.ds(_VB // 2, 32)]
        )


@functools.partial(
    pl.kernel,
    mesh=_mesh,
    compiler_params=_tiled,
    out_type=jax.ShapeDtypeStruct((_FIELDS, _DIM, _BATCH), jnp.float32),
    scratch_types=[
        pltpu.VMEM((8, 128), jnp.int32),
        pltpu.VMEM((128,), jnp.int32),
        pltpu.VMEM((128, 128), jnp.float32),
        pltpu.VMEM((64, 128), jnp.float32),
        pltpu.SemaphoreType.DMA,
    ],
)
def _gth(idx_hbm, tbl_hbm, out_hbm, idxg, jbuf, rows, obuf, sem):
    wid = lax.axis_index("s") * _NC + lax.axis_index("c")
    iota = _iota16()
    bvs = [16 * t + iota for t in range(8)]

    def group(gi, carry):
        g0 = wid * _CPW + 8 * gi
        pltpu.sync_copy(idx_hbm.at[pl.ds(g0, 8)], idxg)

        def chunk(gg, c2):
            c = g0 + gg
            f = c // 128
            bblk = c % 128
            for t in range(8):
                r = idxg[gg, pl.ds(16 * t, 16)]
                jbuf[pl.ds(16 * t, 16)] = r >> 1
            pltpu.async_copy(tbl_hbm.at[jbuf], rows, sem).wait()
            for t in range(8):
                r = idxg[gg, pl.ds(16 * t, 16)]
                h = (r & 1) << 6
                for d in range(64):
                    col = h + d
                    v = plsc.load_gather(rows, [bvs[t], col])
                    obuf[d, pl.ds(16 * t, 16)] = v
            pltpu.sync_copy(obuf, out_hbm.at[f, :, pl.ds(128 * bblk, 128)])
            return c2

        lax.fori_loop(0, 8, chunk, 0)
        return carry

    lax.fori_loop(0, _CPW // 8, group, 0)


def kernel(input, weight):
    wT = weight.T
    wtail = jnp.pad(lax.slice(weight, (_VB, 0), (_V, _DIM)), ((0, 0), (0, 64)))
    tbl = _fmt(wT, wtail)
    idx = input.T.reshape(_NCH, 128).astype(jnp.int32)
    outT = _gth(idx, tbl)
    return outT.transpose(2, 0, 1)


# speedup vs baseline: 2.0421x; 2.0421x over previous
import functools

import jax
import jax.numpy as jnp
from jax import lax
from jax.experimental import pallas as pl
from jax.experimental.pallas import tpu as pltpu
from jax.experimental.pallas import tpu_sc as plsc

_BATCH = 16384
_FIELDS = 26
_DIM = 64
_V = 1000000
_B = _BATCH * _FIELDS
_NC = 2
_NS = 16
_NW = _NC * _NS
_NCH = _B // 128
_CPW = _NCH // _NW

_mesh = plsc.VectorSubcoreMesh(core_axis_name="c", subcore_axis_name="s")
_params = pltpu.CompilerParams(use_tc_tiling_on_sc=True, needs_layout_passes=False)


@functools.partial(
    pl.kernel,
    mesh=_mesh,
    compiler_params=_params,
    out_type=jax.ShapeDtypeStruct((_FIELDS, _DIM, _BATCH), jnp.float32),
    scratch_types=[
        pltpu.VMEM((8, 128), jnp.int32),
        [pltpu.VMEM((128,), jnp.int32) for _ in range(2)],
        [pltpu.VMEM((128, 128), jnp.float32) for _ in range(2)],
        pltpu.VMEM((64, 128), jnp.float32),
        pltpu.SemaphoreType.DMA((2,)),
    ],
)
def _gth(idx_hbm, tbl_hbm, out_hbm, idxg, jbufs, rows, obuf, sems):
    wid = lax.axis_index("s") * _NC + lax.axis_index("c")
    iota = lax.iota(jnp.int32, 16)

    def fire(gg, b):
        for t in range(8):
            r = idxg[gg, pl.ds(16 * t, 16)]
            jbufs[b][pl.ds(16 * t, 16)] = r >> 1
        pltpu.async_copy(tbl_hbm.at[jbufs[b]], rows[b], sems.at[b])

    def drain(c, gg, b):
        f = c // 128
        bblk = c % 128
        pltpu.make_async_copy(tbl_hbm.at[jbufs[b]], rows[b], sems.at[b]).wait()

        def trow(t, c3):
            bv = 16 * t + iota
            r = idxg[gg, pl.ds(16 * t, 16)]
            h = (r & 1) << 6
            for d in range(64):
                v = plsc.load_gather(rows[b], [bv, h + d])
                obuf[d, pl.ds(16 * t, 16)] = v
            return c3

        lax.fori_loop(0, 8, trow, 0)
        pltpu.sync_copy(obuf, out_hbm.at[f, :, pl.ds(128 * bblk, 128)])

    def group(gi, carry):
        g0 = wid * _CPW + 8 * gi
        pltpu.sync_copy(idx_hbm.at[pl.ds(g0, 8)], idxg)
        fire(0, 0)

        def pair(p, c2):
            gg = 2 * p
            fire(gg + 1, 1)
            drain(g0 + gg, gg, 0)
            fire(gg + 2, 0)
            drain(g0 + gg + 1, gg + 1, 1)
            return c2

        lax.fori_loop(0, 3, pair, 0)
        fire(7, 1)
        drain(g0 + 6, 6, 0)
        drain(g0 + 7, 7, 1)
        return carry

    lax.fori_loop(0, _CPW // 8, group, 0)


def kernel(input, weight):
    tbl = weight.reshape(_V // 2, 128)
    idx = input.T.reshape(_NCH, 128).astype(jnp.int32)
    outT = _gth(idx, tbl)
    return outT.transpose(2, 0, 1)


# speedup vs baseline: 2.8090x; 1.3756x over previous
import functools

import jax
import jax.numpy as jnp
from jax import lax
from jax.experimental import pallas as pl
from jax.experimental.pallas import tpu as pltpu
from jax.experimental.pallas import tpu_sc as plsc

_BATCH = 16384
_FIELDS = 26
_DIM = 64
_B = _BATCH * _FIELDS
_NC = 2
_NS = 16
_NW = _NC * _NS
_CHUNK = 256
_ROWS_PER_W = _B // _NW
_CPW = _ROWS_PER_W // _CHUNK

_NBUF = 4

_mesh = plsc.VectorSubcoreMesh(core_axis_name="c", subcore_axis_name="s")


@functools.partial(
    pl.kernel,
    mesh=_mesh,
    compiler_params=pltpu.CompilerParams(use_tc_tiling_on_sc=False),
    out_type=jax.ShapeDtypeStruct((_B, _DIM), jnp.float32),
    scratch_types=[
        pltpu.VMEM((_CPW, _CHUNK), jnp.int32),
        [pltpu.VMEM((_CHUNK, _DIM), jnp.float32) for _ in range(_NBUF)],
        pltpu.SemaphoreType.DMA((_NBUF,)),
    ],
)
def _emb_gather(idx_hbm, table_hbm, out_hbm, idx_v, rows, sems):
    wid = lax.axis_index("s") * _NC + lax.axis_index("c")
    pltpu.sync_copy(idx_hbm.at[pl.ds(wid * _CPW, _CPW)], idx_v)

    def fire(j, b):
        pltpu.async_copy(table_hbm.at[idx_v.at[j]], rows[b], sems.at[b])

    def drain(j, b):
        pltpu.make_async_copy(
            table_hbm.at[idx_v.at[j]], rows[b], sems.at[b]
        ).wait()
        pltpu.sync_copy(
            rows[b], out_hbm.at[pl.ds((wid * _CPW + j) * _CHUNK, _CHUNK)]
        )

    for b in range(_NBUF):
        fire(b, b)

    def group(g, carry):
        for b in range(_NBUF):
            j = g * _NBUF + b
            drain(j, b)
            nxt = j + _NBUF

            @pl.when(nxt < _CPW)
            def _():
                fire(nxt, b)

        return carry

    lax.fori_loop(0, _CPW // _NBUF, group, 0)


def kernel(input, weight):
    idx = input.reshape(_B // _CHUNK, _CHUNK).astype(jnp.int32)
    out = _emb_gather(idx, weight)
    return out.reshape(_BATCH, _FIELDS, _DIM)
